# baseline (speedup 1.0000x reference)
import functools

import jax
import jax.numpy as jnp
from jax import lax
from jax.experimental import pallas as pl
from jax.experimental.pallas import tpu as pltpu
from jax.experimental.pallas import tpu_sc as plsc

M = 2048
K = 1024
E = 8
DFF = 2048
TOP_K = 2
C = (M * TOP_K // E) * 3 // 2
NROWS = E * C
BUF_ROWS = NROWS + 8
FT = 1024
NF = DFF // FT

NC, NS = 2, 16
NW = NC * NS
TPW = M // NW


def _routing_body(rl_ref, slot0_ref, slot1_ref, gsrc0_ref, gsrc1_ref,
                  w0_ref, w1_ref):
    rl = rl_ref[...]
    iota_e = lax.broadcasted_iota(jnp.int32, (M, E), 1)
    m1 = jnp.max(rl, axis=1, keepdims=True)
    id1 = jnp.min(jnp.where(rl == m1, iota_e, E), axis=1, keepdims=True)
    masked = jnp.where(iota_e == id1, -jnp.inf, rl)
    m2 = jnp.max(masked, axis=1, keepdims=True)
    id2 = jnp.min(jnp.where(masked == m2, iota_e, E), axis=1, keepdims=True)
    d = jnp.exp(m2 - m1)
    w0 = 1.0 / (1.0 + d)
    w1 = d / (1.0 + d)
    oh1 = (iota_e == id1).astype(jnp.float32)
    oh2 = (iota_e == id2).astype(jnp.float32)
    tri = (lax.broadcasted_iota(jnp.int32, (M, M), 0)
           > lax.broadcasted_iota(jnp.int32, (M, M), 1))
    c12 = lax.dot_general(tri.astype(jnp.bfloat16),
                          (oh1 + oh2).astype(jnp.bfloat16),
                          (((1,), (0,)), ((), ())),
                          preferred_element_type=jnp.float32)
    pos0 = jnp.sum(c12 * oh1, axis=1, keepdims=True).astype(jnp.int32)
    pos1 = jnp.sum(c12 * oh2, axis=1, keepdims=True).astype(jnp.int32)
    keep0 = pos0 < C
    keep1 = pos1 < C
    slot0_ref[...] = jnp.where(keep0, id1 * C + pos0, NROWS)
    slot1_ref[...] = jnp.where(keep1, id2 * C + pos1, NROWS)
    gsrc0_ref[...] = id1 * C + jnp.minimum(pos0, C - 1)
    gsrc1_ref[...] = id2 * C + jnp.minimum(pos1, C - 1)
    w0_ref[...] = jnp.broadcast_to(jnp.where(keep0, w0, 0.0), (M, 16))
    w1_ref[...] = jnp.broadcast_to(jnp.where(keep1, w1, 0.0), (M, 16))


def _routing(router_logits):
    i32col = jax.ShapeDtypeStruct((M, 1), jnp.int32)
    f32spl = jax.ShapeDtypeStruct((M, 16), jnp.float32)
    return pl.pallas_call(
        _routing_body,
        out_shape=(i32col, i32col, i32col, i32col, f32spl, f32spl),
    )(router_logits)


def _dispatch_body(x_hbm, slot0_hbm, slot1_hbm, buf_hbm, xv, i0, i1, sem):
    wid = lax.axis_index("s") * NC + lax.axis_index("c")
    base = wid * TPW
    pltpu.sync_copy(x_hbm.at[pl.ds(base, TPW)], xv)
    pltpu.sync_copy(slot0_hbm.at[pl.ds(base, TPW)], i0)
    pltpu.sync_copy(slot1_hbm.at[pl.ds(base, TPW)], i1)
    cp0 = pltpu.async_copy(xv, buf_hbm.at[i0], sem)
    cp0.wait()
    cp1 = pltpu.async_copy(xv, buf_hbm.at[i1], sem)
    cp1.wait()


def _dispatch(x, slot0, slot1):
    mesh = plsc.VectorSubcoreMesh(core_axis_name="c", subcore_axis_name="s")
    fn = functools.partial(
        pl.kernel, mesh=mesh,
        out_type=jax.ShapeDtypeStruct((BUF_ROWS, K), jnp.float32),
        scratch_types=[
            pltpu.VMEM((TPW, K), jnp.float32),
            pltpu.VMEM((TPW,), jnp.int32),
            pltpu.VMEM((TPW,), jnp.int32),
            pltpu.SemaphoreType.DMA,
        ],
    )(_dispatch_body)
    return fn(x, slot0, slot1)


def _experts_body(buf_ref, w1g_ref, w1u_ref, w2_ref, out_ref):
    f = pl.program_id(1)
    xb = buf_ref[...].astype(jnp.bfloat16)
    gw = w1g_ref[0].astype(jnp.bfloat16)
    uw = w1u_ref[0].astype(jnp.bfloat16)
    g = lax.dot_general(xb, gw, (((1,), (1,)), ((), ())),
                        preferred_element_type=jnp.float32)
    u = lax.dot_general(xb, uw, (((1,), (1,)), ((), ())),
                        preferred_element_type=jnp.float32)
    a = (g * jax.nn.sigmoid(g) * u).astype(jnp.bfloat16)
    w2b = w2_ref[0].astype(jnp.bfloat16)
    o = lax.dot_general(a, w2b, (((1,), (1,)), ((), ())),
                        preferred_element_type=jnp.float32)

    @pl.when(f == 0)
    def _():
        out_ref[...] = o

    @pl.when(f != 0)
    def _():
        out_ref[...] += o


def _experts(buf, w1, w2):
    return pl.pallas_call(
        _experts_body,
        grid=(E, NF),
        in_specs=[
            pl.BlockSpec((C, K), lambda e, f: (e, 0)),
            pl.BlockSpec((1, FT, K), lambda e, f: (e, f, 0)),
            pl.BlockSpec((1, FT, K), lambda e, f: (e, f + NF, 0)),
            pl.BlockSpec((1, K, FT), lambda e, f: (e, 0, f)),
        ],
        out_specs=pl.BlockSpec((C, K), lambda e, f: (e, 0)),
        out_shape=jax.ShapeDtypeStruct((NROWS, K), jnp.float32),
        compiler_params=pltpu.CompilerParams(
            dimension_semantics=("arbitrary", "arbitrary")),
    )(buf, w1, w1, w2)


def _finalize_body(ob_hbm, g0_hbm, g1_hbm, w0_hbm, w1_hbm, out_hbm,
                   rv0, rv1, gv0, gv1, wv0, wv1, sem):
    wid = lax.axis_index("s") * NC + lax.axis_index("c")
    half = TPW // 2
    ng = K // 16
    for chunk in range(2):
        base = wid * TPW + chunk * half
        pltpu.sync_copy(g0_hbm.at[pl.ds(base, half)], gv0)
        pltpu.sync_copy(g1_hbm.at[pl.ds(base, half)], gv1)
        pltpu.sync_copy(w0_hbm.at[pl.ds(base, half)], wv0)
        pltpu.sync_copy(w1_hbm.at[pl.ds(base, half)], wv1)
        cp0 = pltpu.async_copy(ob_hbm.at[gv0], rv0, sem)
        cp0.wait()
        cp1 = pltpu.async_copy(ob_hbm.at[gv1], rv1, sem)
        cp1.wait()

        def row_body(r, _):
            w0 = wv0[r]
            w1 = wv1[r]

            def grp_body(g, _):
                rv0[r, pl.ds(g * 16, 16)] = (
                    rv0[r, pl.ds(g * 16, 16)] * w0
                    + rv1[r, pl.ds(g * 16, 16)] * w1)
                return 0

            return lax.fori_loop(0, ng, grp_body, 0, unroll=8)

        lax.fori_loop(0, half, row_body, 0)
        pltpu.sync_copy(rv0, out_hbm.at[pl.ds(base, half)])


def _finalize(out_buf, gsrc0, gsrc1, w0c, w1c):
    mesh = plsc.VectorSubcoreMesh(core_axis_name="c", subcore_axis_name="s")
    half = TPW // 2
    fn = functools.partial(
        pl.kernel, mesh=mesh,
        out_type=jax.ShapeDtypeStruct((M, K), jnp.float32),
        scratch_types=[
            pltpu.VMEM((half, K), jnp.float32),
            pltpu.VMEM((half, K), jnp.float32),
            pltpu.VMEM((half,), jnp.int32),
            pltpu.VMEM((half,), jnp.int32),
            pltpu.VMEM((half, 16), jnp.float32),
            pltpu.VMEM((half, 16), jnp.float32),
            pltpu.SemaphoreType.DMA,
        ],
    )(_finalize_body)
    return fn(out_buf, gsrc0, gsrc1, w0c, w1c)


def kernel(x, router_logits, w1, w2):
    slot0, slot1, gsrc0, gsrc1, w0c, w1c = _routing(router_logits)
    buf = _dispatch(x, slot0.reshape(M), slot1.reshape(M))
    out_buf = _experts(buf, w1, w2)
    return _finalize(out_buf, gsrc0.reshape(M), gsrc1.reshape(M), w0c, w1c)


# speedup vs baseline: 1.0987x; 1.0987x over previous
import functools

import jax
import jax.numpy as jnp
from jax import lax
from jax.experimental import pallas as pl
from jax.experimental.pallas import tpu as pltpu
from jax.experimental.pallas import tpu_sc as plsc

M = 2048
K = 1024
E = 8
DFF = 2048
TOP_K = 2
C = (M * TOP_K // E) * 3 // 2
NROWS = E * C
BUF_ROWS = NROWS + 8
FT = 1024
NF = DFF // FT

NC, NS = 2, 16
NW = NC * NS
TPW = M // NW


def _routing_body(rl_ref, slot0_ref, slot1_ref, gsrc0_ref, gsrc1_ref,
                  w0_ref, w1_ref):
    rl = rl_ref[...]
    iota_e = lax.broadcasted_iota(jnp.int32, (M, E), 1)
    m1 = jnp.max(rl, axis=1, keepdims=True)
    id1 = jnp.min(jnp.where(rl == m1, iota_e, E), axis=1, keepdims=True)
    masked = jnp.where(iota_e == id1, -jnp.inf, rl)
    m2 = jnp.max(masked, axis=1, keepdims=True)
    id2 = jnp.min(jnp.where(masked == m2, iota_e, E), axis=1, keepdims=True)
    d = jnp.exp(m2 - m1)
    w0 = 1.0 / (1.0 + d)
    w1 = d / (1.0 + d)
    oh1 = (iota_e == id1).astype(jnp.float32)
    oh2 = (iota_e == id2).astype(jnp.float32)
    tri = (lax.broadcasted_iota(jnp.int32, (M, M), 0)
           > lax.broadcasted_iota(jnp.int32, (M, M), 1))
    c12 = lax.dot_general(tri.astype(jnp.bfloat16),
                          (oh1 + oh2).astype(jnp.bfloat16),
                          (((1,), (0,)), ((), ())),
                          preferred_element_type=jnp.float32)
    pos0 = jnp.sum(c12 * oh1, axis=1, keepdims=True).astype(jnp.int32)
    pos1 = jnp.sum(c12 * oh2, axis=1, keepdims=True).astype(jnp.int32)
    keep0 = pos0 < C
    keep1 = pos1 < C
    slot0_ref[...] = jnp.where(keep0, id1 * C + pos0, NROWS)
    slot1_ref[...] = jnp.where(keep1, id2 * C + pos1, NROWS)
    gsrc0_ref[...] = id1 * C + jnp.minimum(pos0, C - 1)
    gsrc1_ref[...] = id2 * C + jnp.minimum(pos1, C - 1)
    w0_ref[...] = jnp.broadcast_to(jnp.where(keep0, w0, 0.0), (M, 16))
    w1_ref[...] = jnp.broadcast_to(jnp.where(keep1, w1, 0.0), (M, 16))


def _routing(router_logits):
    i32col = jax.ShapeDtypeStruct((M, 1), jnp.int32)
    f32spl = jax.ShapeDtypeStruct((M, 16), jnp.float32)
    return pl.pallas_call(
        _routing_body,
        out_shape=(i32col, i32col, i32col, i32col, f32spl, f32spl),
    )(router_logits)


def _dispatch_body(x_hbm, slot0_hbm, slot1_hbm, buf_hbm, xv, i0, i1, sem):
    wid = lax.axis_index("s") * NC + lax.axis_index("c")
    base = wid * TPW
    pltpu.sync_copy(x_hbm.at[pl.ds(base, TPW)], xv)
    pltpu.sync_copy(slot0_hbm.at[pl.ds(base, TPW)], i0)
    pltpu.sync_copy(slot1_hbm.at[pl.ds(base, TPW)], i1)
    cp0 = pltpu.async_copy(xv, buf_hbm.at[i0], sem)
    cp0.wait()
    cp1 = pltpu.async_copy(xv, buf_hbm.at[i1], sem)
    cp1.wait()


def _dispatch(x, slot0, slot1):
    mesh = plsc.VectorSubcoreMesh(core_axis_name="c", subcore_axis_name="s")
    fn = functools.partial(
        pl.kernel, mesh=mesh,
        out_type=jax.ShapeDtypeStruct((BUF_ROWS, K), jnp.float32),
        scratch_types=[
            pltpu.VMEM((TPW, K), jnp.float32),
            pltpu.VMEM((TPW,), jnp.int32),
            pltpu.VMEM((TPW,), jnp.int32),
            pltpu.SemaphoreType.DMA,
        ],
    )(_dispatch_body)
    return fn(x, slot0, slot1)


def _experts_body(buf_ref, w1g_ref, w1u_ref, w2_ref, out_ref):
    f = pl.program_id(1)
    xb = buf_ref[...].astype(jnp.bfloat16)
    gw = w1g_ref[0].astype(jnp.bfloat16)
    uw = w1u_ref[0].astype(jnp.bfloat16)
    g = lax.dot_general(xb, gw, (((1,), (1,)), ((), ())),
                        preferred_element_type=jnp.float32)
    u = lax.dot_general(xb, uw, (((1,), (1,)), ((), ())),
                        preferred_element_type=jnp.float32)
    a = (g * jax.nn.sigmoid(g) * u).astype(jnp.bfloat16)
    w2b = w2_ref[0].astype(jnp.bfloat16)
    o = lax.dot_general(a, w2b, (((1,), (1,)), ((), ())),
                        preferred_element_type=jnp.float32)

    @pl.when(f == 0)
    def _():
        out_ref[...] = o

    @pl.when(f != 0)
    def _():
        out_ref[...] += o


def _experts(buf, w1, w2):
    return pl.pallas_call(
        _experts_body,
        grid=(E, NF),
        in_specs=[
            pl.BlockSpec((C, K), lambda e, f: (e, 0)),
            pl.BlockSpec((1, FT, K), lambda e, f: (e, f, 0)),
            pl.BlockSpec((1, FT, K), lambda e, f: (e, f + NF, 0)),
            pl.BlockSpec((1, K, FT), lambda e, f: (e, 0, f)),
        ],
        out_specs=pl.BlockSpec((C, K), lambda e, f: (e, 0)),
        out_shape=jax.ShapeDtypeStruct((NROWS, K), jnp.float32),
        compiler_params=pltpu.CompilerParams(
            dimension_semantics=("arbitrary", "arbitrary")),
    )(buf, w1, w1, w2)


def _finalize_body(ob_hbm, g0_hbm, g1_hbm, w0_hbm, w1_hbm, out_hbm,
                   rv0, rv1, gv0, gv1, wv0, wv1, sem):
    wid = lax.axis_index("s") * NC + lax.axis_index("c")
    half = TPW // 2
    ng = K // 16
    for chunk in range(2):
        base = wid * TPW + chunk * half
        pltpu.sync_copy(g0_hbm.at[pl.ds(base, half)], gv0)
        pltpu.sync_copy(g1_hbm.at[pl.ds(base, half)], gv1)
        pltpu.sync_copy(w0_hbm.at[pl.ds(base, half)], wv0)
        pltpu.sync_copy(w1_hbm.at[pl.ds(base, half)], wv1)
        cp0 = pltpu.async_copy(ob_hbm.at[gv0], rv0, sem)
        cp1 = pltpu.async_copy(ob_hbm.at[gv1], rv1, sem)
        cp0.wait()
        cp1.wait()

        def row_body(r, _):
            w0 = wv0[r]
            w1 = wv1[r]
            for g in range(ng):
                rv0[r, pl.ds(g * 16, 16)] = (
                    rv0[r, pl.ds(g * 16, 16)] * w0
                    + rv1[r, pl.ds(g * 16, 16)] * w1)
            return 0

        lax.fori_loop(0, half, row_body, 0)
        pltpu.sync_copy(rv0, out_hbm.at[pl.ds(base, half)])


def _finalize(out_buf, gsrc0, gsrc1, w0c, w1c):
    mesh = plsc.VectorSubcoreMesh(core_axis_name="c", subcore_axis_name="s")
    half = TPW // 2
    fn = functools.partial(
        pl.kernel, mesh=mesh,
        out_type=jax.ShapeDtypeStruct((M, K), jnp.float32),
        scratch_types=[
            pltpu.VMEM((half, K), jnp.float32),
            pltpu.VMEM((half, K), jnp.float32),
            pltpu.VMEM((half,), jnp.int32),
            pltpu.VMEM((half,), jnp.int32),
            pltpu.VMEM((half, 16), jnp.float32),
            pltpu.VMEM((half, 16), jnp.float32),
            pltpu.SemaphoreType.DMA,
        ],
    )(_finalize_body)
    return fn(out_buf, gsrc0, gsrc1, w0c, w1c)


def kernel(x, router_logits, w1, w2):
    slot0, slot1, gsrc0, gsrc1, w0c, w1c = _routing(router_logits)
    buf = _dispatch(x, slot0.reshape(M), slot1.reshape(M))
    out_buf = _experts(buf, w1, w2)
    return _finalize(out_buf, gsrc0.reshape(M), gsrc1.reshape(M), w0c, w1c)
